# baseline (speedup 1.0000x reference)
import jax
import jax.numpy as jnp
from jax import lax
from jax.experimental import pallas as pl
from jax.experimental.pallas import tpu as pltpu
from jax.experimental.pallas import tpu_sc as plsc

N_NODES = 10000
N_EDGES = 160000
D_FEAT = 256
DH = D_FEAT // 2
NS = 16
EPT = N_EDGES // NS
CHUNK = 80
NCHUNK = EPT // CHUNK
N_PAD = 10240
RPT = N_PAD // NS
BLK = 80
NBLK = RPT // BLK
NROW = 3
NIDX = 6


def _body(x, srcr, dstr, w, out,
          rb0, rb1, rb2,
          sb0, sb1, sb2, sb3, sb4, sb5,
          db0, db1, db2, db3, db4, db5,
          wb0, wb1, wb2, wb3, wb4, wb5,
          rsb,
          out_sh, rs_sh,
          gs0, gs1, gs2, is0, is1, is2, is3, is4, is5,
          ss0, ss1, ss2, rssem):
    c = lax.axis_index("c")
    s = lax.axis_index("s")
    rows = [rb0, rb1, rb2]
    sbufs = [sb0, sb1, sb2, sb3, sb4, sb5]
    dbufs = [db0, db1, db2, db3, db4, db5]
    wbufs = [wb0, wb1, wb2, wb3, wb4, wb5]
    gsems = [gs0, gs1, gs2]
    isems = [is0, is1, is2, is3, is4, is5]
    ssems = [ss0, ss1, ss2]

    zero16 = jnp.zeros((16,), jnp.float32)
    ebase = s * EPT
    row0 = s * RPT

    def zrow(i, _):
        for j in range(DH // 16):
            rb0[i, pl.ds(16 * j, 16)] = zero16
        return 0
    lax.fori_loop(0, BLK, zrow, 0)

    def zrs(i, _):
        rsb[pl.ds(i * 16, 16)] = zero16
        return 0
    lax.fori_loop(0, RPT // 16, zrs, 0)

    for k in range(NBLK):
        pltpu.sync_copy(rb0, out_sh.at[pl.ds(row0 + k * BLK, BLK), :])
    pltpu.sync_copy(rsb, rs_sh.at[pl.ds(row0, RPT)])
    plsc.subcore_barrier()

    def scale(rb, wbuf):
        for g in range(CHUNK // 16):
            wv = wbuf[pl.ds(g * 16, 16)]
            for e in range(16):
                ws = wv[e]
                r = g * 16 + e
                for cj in range(DH // 16):
                    sl = pl.ds(16 * cj, 16)
                    rb[r, sl] = rb[r, sl] * ws

    def issue_idx(j, u):
        b = ebase + j * CHUNK
        pltpu.async_copy(srcr.at[pl.ds(b, CHUNK)], sbufs[u], isems[u])
        pltpu.async_copy(dstr.at[pl.ds(b, CHUNK)], dbufs[u], isems[u])
        pltpu.async_copy(w.at[pl.ds(b, CHUNK)], wbufs[u], isems[u])

    def wait_idx(j, u):
        b = ebase + j * CHUNK
        pltpu.make_async_copy(srcr.at[pl.ds(b, CHUNK)], sbufs[u], isems[u]).wait()
        pltpu.make_async_copy(dstr.at[pl.ds(b, CHUNK)], dbufs[u], isems[u]).wait()
        pltpu.make_async_copy(w.at[pl.ds(b, CHUNK)], wbufs[u], isems[u]).wait()

    def issue_gather(u, b):
        pltpu.async_copy(x.at[sbufs[u], pl.ds(c * DH, DH)], rows[b], gsems[b])

    def wait_gather(u, b):
        pltpu.make_async_copy(x.at[sbufs[u], pl.ds(c * DH, DH)], rows[b],
                              gsems[b]).wait()

    def issue_scatter(u, b):
        pltpu.async_copy(rows[b], out_sh.at[dbufs[u]], ssems[b], add=True)

    def wait_scatter(u, b):
        pltpu.make_async_copy(rows[b], out_sh.at[dbufs[u]], ssems[b]).wait()

    def issue_rs(u):
        pltpu.async_copy(wbufs[u], rs_sh.at[dbufs[u]], rssem, add=True)

    def wait_rs():
        pltpu.make_async_copy(wb0, rs_sh.at[db0], rssem).wait()

    issue_idx(0, 0)
    issue_idx(1, 1)
    issue_idx(2, 2)
    wait_idx(0, 0)
    issue_gather(0, 0)

    def step(t, _):
        for u in range(NIDX):
            j = NIDX * t + u
            b = u % NROW

            @pl.when(jnp.logical_and(j >= 2, j - 2 < NCHUNK))
            def _():
                wait_scatter((u + 4) % NIDX, (b + 1) % NROW)

            @pl.when(j + 1 < NCHUNK)
            def _():
                wait_idx(j + 1, (u + 1) % NIDX)
                issue_gather((u + 1) % NIDX, (b + 1) % NROW)

            @pl.when(j < NCHUNK)
            def _():
                wait_gather(u, b)
                scale(rows[b], wbufs[u])
                issue_scatter(u, b)
                issue_rs(u)

            @pl.when(jnp.logical_and(j >= 3, j - 3 < NCHUNK))
            def _():
                wait_rs()

            @pl.when(j + 3 < NCHUNK)
            def _():
                issue_idx(j + 3, (u + 3) % NIDX)
        return 0

    lax.fori_loop(0, NCHUNK // NIDX + 2, step, 0)
    plsc.subcore_barrier()

    pltpu.sync_copy(rs_sh.at[pl.ds(row0, RPT)], rsb)

    def inv_chunk(i, _):
        sl = pl.ds(i * 16, 16)
        rsb[sl] = 1.0 / jnp.maximum(rsb[sl], 1e-12)
        return 0
    lax.fori_loop(0, RPT // 16, inv_chunk, 0)

    stage = rows[0]
    for k in range(NBLK):
        r0 = row0 + k * BLK
        pltpu.sync_copy(out_sh.at[pl.ds(r0, BLK), :], stage)

        def ngrp(g, _):
            ivv = rsb[pl.ds(k * BLK + g * 16, 16)]
            for e in range(16):
                ive = ivv[e]
                r = g * 16 + e
                for cj in range(DH // 16):
                    sl = pl.ds(16 * cj, 16)
                    stage[r, sl] = stage[r, sl] * ive
            return 0
        lax.fori_loop(0, BLK // 16, ngrp, 0)

        @pl.when(jnp.logical_or(s < NS - 1, k < 5))
        def _():
            pltpu.sync_copy(stage, out.at[pl.ds(r0, BLK), pl.ds(c * DH, DH)])


def _make_kernel():
    mesh = plsc.VectorSubcoreMesh(core_axis_name="c", subcore_axis_name="s")
    row_buf = pltpu.VMEM((CHUNK, DH), jnp.float32)
    ibuf = pltpu.VMEM((CHUNK,), jnp.int32)
    fbuf = pltpu.VMEM((CHUNK,), jnp.float32)
    sem = pltpu.SemaphoreType.DMA
    return pl.kernel(
        _body,
        out_type=jax.ShapeDtypeStruct((N_NODES, D_FEAT), jnp.float32),
        mesh=mesh,
        scratch_types=(
            [row_buf] * NROW
            + [ibuf] * NIDX
            + [ibuf] * NIDX
            + [fbuf] * NIDX
            + [pltpu.VMEM((RPT,), jnp.float32)]
            + [pltpu.VMEM_SHARED((N_PAD, DH), jnp.float32),
               pltpu.VMEM_SHARED((N_PAD,), jnp.float32)]
            + [sem] * (NROW + NIDX + NROW + 1)
        ),
    )


@jax.jit
def kernel(x, edge_index, edge_weight):
    return _make_kernel()(x, edge_index[0], edge_index[1], edge_weight)


# speedup vs baseline: 1.3881x; 1.3881x over previous
import jax
import jax.numpy as jnp
from jax import lax
from jax.experimental import pallas as pl
from jax.experimental.pallas import tpu as pltpu
from jax.experimental.pallas import tpu_sc as plsc

N_NODES = 10000
N_EDGES = 160000
D_FEAT = 256
DH = D_FEAT // 2
NS = 16
EPT = N_EDGES // NS
CHUNK = 80
NCHUNK = EPT // CHUNK
N_PAD = 10240
RPT = N_PAD // NS
BLK = 80
NBLK = RPT // BLK
NROW = 3
NIDX = 6


def _body(x, srcr, dstr, w, out,
          rb0, rb1, rb2,
          sb0, sb1, sb2, sb3, sb4, sb5,
          db0, db1, db2, db3, db4, db5,
          wb0, wb1, wb2, wb3, wb4, wb5,
          rsb,
          out_sh, rs_sh,
          gs0, gs1, gs2, is0, is1, is2, is3, is4, is5,
          ss0, ss1, ss2, rssem):
    c = lax.axis_index("c")
    s = lax.axis_index("s")
    rows = [rb0, rb1, rb2]
    sbufs = [sb0, sb1, sb2, sb3, sb4, sb5]
    dbufs = [db0, db1, db2, db3, db4, db5]
    wbufs = [wb0, wb1, wb2, wb3, wb4, wb5]
    gsems = [gs0, gs1, gs2]
    isems = [is0, is1, is2, is3, is4, is5]
    ssems = [ss0, ss1, ss2]

    zero16 = jnp.zeros((16,), jnp.float32)
    ebase = s * EPT
    row0 = s * RPT

    def scale(rb, wbuf):
        def grp(g, _):
            wv = wbuf[pl.ds(g * 16, 16)]
            for e in range(16):
                ws = wv[e]
                r = g * 16 + e
                for cj in range(DH // 16):
                    sl = pl.ds(16 * cj, 16)
                    rb[r, sl] = rb[r, sl] * ws
            return 0
        lax.fori_loop(0, CHUNK // 16, grp, 0)

    def issue_idx(j, u):
        b = ebase + j * CHUNK
        pltpu.async_copy(srcr.at[pl.ds(b, CHUNK)], sbufs[u], isems[u])
        pltpu.async_copy(dstr.at[pl.ds(b, CHUNK)], dbufs[u], isems[u])
        pltpu.async_copy(w.at[pl.ds(b, CHUNK)], wbufs[u], isems[u])

    def wait_idx(j, u):
        b = ebase + j * CHUNK
        pltpu.make_async_copy(srcr.at[pl.ds(b, CHUNK)], sbufs[u], isems[u]).wait()
        pltpu.make_async_copy(dstr.at[pl.ds(b, CHUNK)], dbufs[u], isems[u]).wait()
        pltpu.make_async_copy(w.at[pl.ds(b, CHUNK)], wbufs[u], isems[u]).wait()

    def issue_gather(u, b):
        pltpu.async_copy(x.at[sbufs[u], pl.ds(c * DH, DH)], rows[b], gsems[b])

    def wait_gather(u, b):
        pltpu.make_async_copy(x.at[sbufs[u], pl.ds(c * DH, DH)], rows[b],
                              gsems[b]).wait()

    def issue_scatter(u, b):
        pltpu.async_copy(rows[b], out_sh.at[dbufs[u]], ssems[b], add=True)

    def wait_scatter(u, b):
        pltpu.make_async_copy(rows[b], out_sh.at[dbufs[u]], ssems[b]).wait()

    def issue_rs(u):
        pltpu.async_copy(wbufs[u], rs_sh.at[dbufs[u]], rssem, add=True)

    def wait_rs():
        pltpu.make_async_copy(wb0, rs_sh.at[db0], rssem).wait()

    issue_idx(0, 0)
    issue_idx(1, 1)
    issue_idx(2, 2)

    def zrow(i, _):
        for j in range(DH // 16):
            rb1[i, pl.ds(16 * j, 16)] = zero16
        return 0
    lax.fori_loop(0, BLK, zrow, 0)

    def zrs(i, _):
        rsb[pl.ds(i * 16, 16)] = zero16
        return 0
    lax.fori_loop(0, RPT // 16, zrs, 0)

    for k in range(NBLK):
        pltpu.async_copy(rb1, out_sh.at[pl.ds(row0 + k * BLK, BLK), :], ss0)
    pltpu.async_copy(rsb, rs_sh.at[pl.ds(row0, RPT)], rssem)

    wait_idx(0, 0)
    issue_gather(0, 0)

    for k in range(NBLK):
        pltpu.make_async_copy(rb1, out_sh.at[pl.ds(row0 + k * BLK, BLK), :],
                              ss0).wait()
    pltpu.make_async_copy(rsb, rs_sh.at[pl.ds(row0, RPT)], rssem).wait()
    plsc.subcore_barrier()


    def step(t, _):
        for u in range(NIDX):
            j = NIDX * t + u
            b = u % NROW

            @pl.when(jnp.logical_and(j >= 2, j - 2 < NCHUNK))
            def _():
                wait_scatter((u + 4) % NIDX, (b + 1) % NROW)

            @pl.when(j + 1 < NCHUNK)
            def _():
                wait_idx(j + 1, (u + 1) % NIDX)
                issue_gather((u + 1) % NIDX, (b + 1) % NROW)

            @pl.when(j < NCHUNK)
            def _():
                wait_gather(u, b)
                scale(rows[b], wbufs[u])
                issue_scatter(u, b)
                issue_rs(u)

            @pl.when(jnp.logical_and(j >= 3, j - 3 < NCHUNK))
            def _():
                wait_rs()

            @pl.when(j + 3 < NCHUNK)
            def _():
                issue_idx(j + 3, (u + 3) % NIDX)
        return 0

    lax.fori_loop(0, NCHUNK // NIDX + 2, step, 0)
    plsc.subcore_barrier()

    def issue_in(k):
        pltpu.async_copy(out_sh.at[pl.ds(row0 + k * BLK, BLK), :],
                         rows[k % NROW], gsems[k % NROW])

    def wait_in(k):
        pltpu.make_async_copy(out_sh.at[pl.ds(row0 + k * BLK, BLK), :],
                              rows[k % NROW], gsems[k % NROW]).wait()

    def issue_out(k):
        pltpu.async_copy(rows[k % NROW],
                         out.at[pl.ds(row0 + k * BLK, BLK), pl.ds(c * DH, DH)],
                         ssems[k % NROW])

    def wait_out(k):
        pltpu.make_async_copy(
            rows[k % NROW],
            out.at[pl.ds(row0 + k * BLK, BLK), pl.ds(c * DH, DH)],
            ssems[k % NROW]).wait()

    def real(k):
        return jnp.logical_or(s < NS - 1, k < 5)

    issue_in(0)
    issue_in(1)
    pltpu.sync_copy(rs_sh.at[pl.ds(row0, RPT)], rsb)

    def inv_chunk(i, _):
        sl = pl.ds(i * 16, 16)
        rsb[sl] = 1.0 / jnp.maximum(rsb[sl], 1e-12)
        return 0
    lax.fori_loop(0, RPT // 16, inv_chunk, 0)

    for k in range(NBLK):
        stage = rows[k % NROW]
        wait_in(k)

        def ngrp(g, _):
            ivv = rsb[pl.ds(k * BLK + g * 16, 16)]
            for e in range(16):
                ive = ivv[e]
                r = g * 16 + e
                for cj in range(DH // 16):
                    sl = pl.ds(16 * cj, 16)
                    stage[r, sl] = stage[r, sl] * ive
            return 0
        lax.fori_loop(0, BLK // 16, ngrp, 0)

        @pl.when(real(k))
        def _():
            issue_out(k)

        if k + 2 < NBLK:
            if k >= 1:
                @pl.when(real(k - 1))
                def _():
                    wait_out(k - 1)
            issue_in(k + 2)

    for k in range(NBLK - 3, NBLK):
        @pl.when(real(k))
        def _():
            wait_out(k)


def _make_kernel():
    mesh = plsc.VectorSubcoreMesh(core_axis_name="c", subcore_axis_name="s")
    row_buf = pltpu.VMEM((CHUNK, DH), jnp.float32)
    ibuf = pltpu.VMEM((CHUNK,), jnp.int32)
    fbuf = pltpu.VMEM((CHUNK,), jnp.float32)
    sem = pltpu.SemaphoreType.DMA
    return pl.kernel(
        _body,
        out_type=jax.ShapeDtypeStruct((N_NODES, D_FEAT), jnp.float32),
        mesh=mesh,
        scratch_types=(
            [row_buf] * NROW
            + [ibuf] * NIDX
            + [ibuf] * NIDX
            + [fbuf] * NIDX
            + [pltpu.VMEM((RPT,), jnp.float32)]
            + [pltpu.VMEM_SHARED((N_PAD, DH), jnp.float32),
               pltpu.VMEM_SHARED((N_PAD,), jnp.float32)]
            + [sem] * (NROW + NIDX + NROW + 1)
        ),
    )


@jax.jit
def kernel(x, edge_index, edge_weight):
    return _make_kernel()(x, edge_index[0], edge_index[1], edge_weight)


# speedup vs baseline: 1.3973x; 1.0066x over previous
import jax
import jax.numpy as jnp
from jax import lax
from jax.experimental import pallas as pl
from jax.experimental.pallas import tpu as pltpu
from jax.experimental.pallas import tpu_sc as plsc

N_NODES = 10000
N_EDGES = 160000
D_FEAT = 256
DH = D_FEAT // 2
NS = 16
EPT = N_EDGES // NS
CHUNK = 80
NCHUNK = EPT // CHUNK
N_PAD = 10240
RPT = N_PAD // NS
BLK = 80
NBLK = RPT // BLK
NROW = 3
NIDX = 6


def _body(x, srcr, dstr, w, out,
          rb0, rb1, rb2,
          sb0, sb1, sb2, sb3, sb4, sb5,
          db0, db1, db2, db3, db4, db5,
          wb0, wb1, wb2, wb3, wb4, wb5,
          rsb,
          out_sh, rs_sh,
          gs0, gs1, gs2, is0, is1, is2, is3, is4, is5,
          ss0, ss1, ss2, rssem):
    c = lax.axis_index("c")
    s = lax.axis_index("s")
    rows = [rb0, rb1, rb2]
    sbufs = [sb0, sb1, sb2, sb3, sb4, sb5]
    dbufs = [db0, db1, db2, db3, db4, db5]
    wbufs = [wb0, wb1, wb2, wb3, wb4, wb5]
    gsems = [gs0, gs1, gs2]
    isems = [is0, is1, is2, is3, is4, is5]
    ssems = [ss0, ss1, ss2]

    zero16 = jnp.zeros((16,), jnp.float32)
    ebase = s * EPT
    row0 = s * RPT

    def scale(rb, wbuf, ob):
        def grp(g, _):
            wv = wbuf[pl.ds(g * 16, 16)]
            wss = [wv[e] for e in range(16)]
            for e in range(0, 16, 2):
                r0 = g * 16 + e
                r1 = r0 + 1
                for cj in range(DH // 16):
                    sl = pl.ds(16 * cj, 16)
                    ob[r0, sl] = rb[r0, sl] * wss[e]
                    ob[r1, sl] = rb[r1, sl] * wss[e + 1]
            return 0
        lax.fori_loop(0, CHUNK // 16, grp, 0)

    def issue_idx(j, u):
        b = ebase + j * CHUNK
        pltpu.async_copy(srcr.at[pl.ds(b, CHUNK)], sbufs[u], isems[u])
        pltpu.async_copy(dstr.at[pl.ds(b, CHUNK)], dbufs[u], isems[u])
        pltpu.async_copy(w.at[pl.ds(b, CHUNK)], wbufs[u], isems[u])

    def wait_idx(j, u):
        b = ebase + j * CHUNK
        pltpu.make_async_copy(srcr.at[pl.ds(b, CHUNK)], sbufs[u], isems[u]).wait()
        pltpu.make_async_copy(dstr.at[pl.ds(b, CHUNK)], dbufs[u], isems[u]).wait()
        pltpu.make_async_copy(w.at[pl.ds(b, CHUNK)], wbufs[u], isems[u]).wait()

    def issue_gather(u, b):
        pltpu.async_copy(x.at[sbufs[u], pl.ds(c * DH, DH)], rows[b], gsems[b])

    def wait_gather(u, b):
        pltpu.make_async_copy(x.at[sbufs[u], pl.ds(c * DH, DH)], rows[b],
                              gsems[b]).wait()

    def issue_scatter(u, b):
        pltpu.async_copy(rows[b], out_sh.at[dbufs[u]], ssems[b], add=True)

    def wait_scatter(u, b):
        pltpu.make_async_copy(rows[b], out_sh.at[dbufs[u]], ssems[b]).wait()

    def issue_rs(u):
        pltpu.async_copy(wbufs[u], rs_sh.at[dbufs[u]], rssem, add=True)

    def wait_rs():
        pltpu.make_async_copy(wb0, rs_sh.at[db0], rssem).wait()

    issue_idx(0, 0)
    issue_idx(1, 1)
    issue_idx(2, 2)

    def zrow(i, _):
        for j in range(DH // 16):
            rb1[i, pl.ds(16 * j, 16)] = zero16
        return 0
    lax.fori_loop(0, BLK, zrow, 0)

    def zrs(i, _):
        rsb[pl.ds(i * 16, 16)] = zero16
        return 0
    lax.fori_loop(0, RPT // 16, zrs, 0)

    for k in range(NBLK):
        pltpu.async_copy(rb1, out_sh.at[pl.ds(row0 + k * BLK, BLK), :], ss0)
    pltpu.async_copy(rsb, rs_sh.at[pl.ds(row0, RPT)], rssem)

    wait_idx(0, 0)
    issue_gather(0, 0)

    for k in range(NBLK):
        pltpu.make_async_copy(rb1, out_sh.at[pl.ds(row0 + k * BLK, BLK), :],
                              ss0).wait()
    pltpu.make_async_copy(rsb, rs_sh.at[pl.ds(row0, RPT)], rssem).wait()
    plsc.subcore_barrier()


    def step(t, _):
        for u in range(NIDX):
            j = NIDX * t + u
            b = u % NROW

            @pl.when(jnp.logical_and(j >= 2, j - 2 < NCHUNK))
            def _():
                wait_scatter((u + 4) % NIDX, (b + 1) % NROW)

            @pl.when(j + 1 < NCHUNK)
            def _():
                wait_idx(j + 1, (u + 1) % NIDX)
                issue_gather((u + 1) % NIDX, (b + 1) % NROW)

            @pl.when(j < NCHUNK)
            def _():
                wait_gather(u, b)
                scale(rows[b], wbufs[u], rows[b])
                issue_scatter(u, b)
                issue_rs(u)

            @pl.when(jnp.logical_and(j >= 3, j - 3 < NCHUNK))
            def _():
                wait_rs()

            @pl.when(j + 3 < NCHUNK)
            def _():
                issue_idx(j + 3, (u + 3) % NIDX)
        return 0

    lax.fori_loop(0, NCHUNK // NIDX + 2, step, 0)
    plsc.subcore_barrier()

    def issue_in(k):
        pltpu.async_copy(out_sh.at[pl.ds(row0 + k * BLK, BLK), :],
                         rows[k % NROW], gsems[k % NROW])

    def wait_in(k):
        pltpu.make_async_copy(out_sh.at[pl.ds(row0 + k * BLK, BLK), :],
                              rows[k % NROW], gsems[k % NROW]).wait()

    def issue_out(k):
        pltpu.async_copy(rows[k % NROW],
                         out.at[pl.ds(row0 + k * BLK, BLK), pl.ds(c * DH, DH)],
                         ssems[k % NROW])

    def wait_out(k):
        pltpu.make_async_copy(
            rows[k % NROW],
            out.at[pl.ds(row0 + k * BLK, BLK), pl.ds(c * DH, DH)],
            ssems[k % NROW]).wait()

    def real(k):
        return jnp.logical_or(s < NS - 1, k < 5)

    issue_in(0)
    issue_in(1)
    pltpu.sync_copy(rs_sh.at[pl.ds(row0, RPT)], rsb)

    def inv_chunk(i, _):
        sl = pl.ds(i * 16, 16)
        rsb[sl] = 1.0 / jnp.maximum(rsb[sl], 1e-12)
        return 0
    lax.fori_loop(0, RPT // 16, inv_chunk, 0)

    for k in range(NBLK):
        stage = rows[k % NROW]
        wait_in(k)

        def ngrp(g, _):
            ivv = rsb[pl.ds(k * BLK + g * 16, 16)]
            for e in range(16):
                ive = ivv[e]
                r = g * 16 + e
                for cj in range(DH // 16):
                    sl = pl.ds(16 * cj, 16)
                    stage[r, sl] = stage[r, sl] * ive
            return 0
        lax.fori_loop(0, BLK // 16, ngrp, 0)

        @pl.when(real(k))
        def _():
            issue_out(k)

        if k + 2 < NBLK:
            if k >= 1:
                @pl.when(real(k - 1))
                def _():
                    wait_out(k - 1)
            issue_in(k + 2)

    for k in range(NBLK - 3, NBLK):
        @pl.when(real(k))
        def _():
            wait_out(k)


def _make_kernel():
    mesh = plsc.VectorSubcoreMesh(core_axis_name="c", subcore_axis_name="s")
    row_buf = pltpu.VMEM((CHUNK, DH), jnp.float32)
    ibuf = pltpu.VMEM((CHUNK,), jnp.int32)
    fbuf = pltpu.VMEM((CHUNK,), jnp.float32)
    sem = pltpu.SemaphoreType.DMA
    return pl.kernel(
        _body,
        out_type=jax.ShapeDtypeStruct((N_NODES, D_FEAT), jnp.float32),
        mesh=mesh,
        scratch_types=(
            [row_buf] * NROW
            + [ibuf] * NIDX
            + [ibuf] * NIDX
            + [fbuf] * NIDX
            + [pltpu.VMEM((RPT,), jnp.float32)]
            + [pltpu.VMEM_SHARED((N_PAD, DH), jnp.float32),
               pltpu.VMEM_SHARED((N_PAD,), jnp.float32)]
            + [sem] * (NROW + NIDX + NROW + 1)
        ),
    )


@jax.jit
def kernel(x, edge_index, edge_weight):
    return _make_kernel()(x, edge_index[0], edge_index[1], edge_weight)
